# baseline (speedup 1.0000x reference)
import jax
import jax.numpy as jnp
from jax import lax
from jax.experimental import pallas as pl
from jax.experimental.pallas import tpu as pltpu
from jax.experimental.pallas import tpu_sc as plsc

B = 4096
L = 50
VOCAB = 1000000
D = 64

NC = 2
NS = 16
NW = NC * NS
PPW = (B * L) // NW
CH = 128
NCH = PPW // CH
DUMP = B * L


def _body(tok_hbm, seq_hbm, table_hbm, zero_hbm, out_hbm,
          tok_v, seq_v, drow_v, dzero_v, rows_v, zero_v, sem, semz):
    wid = lax.axis_index("s") * NC + lax.axis_index("c")
    base = wid * PPW
    dump = DUMP + wid * CH

    pltpu.sync_copy(tok_hbm.at[pl.ds(base, PPW)], tok_v)
    pltpu.sync_copy(seq_hbm, seq_v)
    pltpu.sync_copy(zero_hbm, zero_v)

    iota = lax.iota(jnp.int32, 16)
    inv_l = jnp.full((16,), 1.0 / L, jnp.float32)

    def compute_chunk(j, _):
        for k in range(CH // 16):
            p = base + j * CH + k * 16 + iota
            q = (p.astype(jnp.float32) * inv_l).astype(jnp.int32)
            q = jnp.where((q + 1) * L <= p, q + 1, q)
            q = jnp.where(q * L > p, q - 1, q)
            l = p - q * L
            seq = plsc.load_gather(seq_v, [q])
            valid = l < seq
            slot = dump + (p & (CH - 1))
            drow_v[j, pl.ds(k * 16, 16)] = jnp.where(valid, p, slot)
            dzero_v[j, pl.ds(k * 16, 16)] = jnp.where(valid, slot, p)
        return 0

    lax.fori_loop(0, NCH, compute_chunk, 0)

    def move_chunk(g, _):
        pltpu.async_copy(
            table_hbm.at[tok_v.at[pl.ds(g * CH, CH)]], rows_v, sem).wait()
        rows_out = pltpu.async_copy(rows_v, out_hbm.at[drow_v.at[g]], sem)
        zero_out = pltpu.async_copy(zero_v, out_hbm.at[dzero_v.at[g]], semz)
        rows_out.wait()
        zero_out.wait()
        return 0

    lax.fori_loop(0, NCH, move_chunk, 0)


def kernel(token_ids, seq_lens, glove_table):
    tok_flat = token_ids.reshape(B * L).astype(jnp.int32)
    seq_lens = seq_lens.astype(jnp.int32)
    zeros_src = jnp.zeros((CH, D), jnp.float32)

    mesh = plsc.VectorSubcoreMesh(core_axis_name="c", subcore_axis_name="s")
    out = pl.kernel(
        _body,
        out_type=jax.ShapeDtypeStruct((B * L + NW * CH, D), jnp.float32),
        mesh=mesh,
        compiler_params=pltpu.CompilerParams(use_tc_tiling_on_sc=False,
                                             needs_layout_passes=False),
        scratch_types=[
            pltpu.VMEM((PPW,), jnp.int32),
            pltpu.VMEM((B,), jnp.int32),
            pltpu.VMEM((NCH, CH), jnp.int32),
            pltpu.VMEM((NCH, CH), jnp.int32),
            pltpu.VMEM((CH, D), jnp.float32),
            pltpu.VMEM((CH, D), jnp.float32),
            pltpu.SemaphoreType.DMA,
            pltpu.SemaphoreType.DMA,
        ],
    )(tok_flat, seq_lens, glove_table, zeros_src)
    return out[:B * L].reshape(B, L, D)


# speedup vs baseline: 1.2559x; 1.2559x over previous
import jax
import jax.numpy as jnp
from jax import lax
from jax.experimental import pallas as pl
from jax.experimental.pallas import tpu as pltpu
from jax.experimental.pallas import tpu_sc as plsc

B = 4096
L = 50
VOCAB = 1000000
D = 64

NC = 2
NS = 16
NW = NC * NS
PPW = (B * L) // NW
CH = 128
NCH = PPW // CH


def _body(tok_hbm, seq_hbm, table_hbm, zero_hbm, out_hbm,
          tok_in, seq_v, vtok, vdst, zdst, rows_v, zero_v,
          sem_g, sem_s, sem_z):
    wid = lax.axis_index("s") * NC + lax.axis_index("c")
    base = wid * PPW

    pltpu.sync_copy(tok_hbm.at[pl.ds(base, PPW)], tok_in)
    pltpu.sync_copy(seq_hbm, seq_v)
    pltpu.sync_copy(zero_hbm, zero_v)

    iota = lax.iota(jnp.int32, 16)
    inv_l = jnp.full((16,), 1.0 / L, jnp.float32)

    def compute_chunk(j, carry):
        nv, ni = carry
        for k in range(CH // 16):
            off = j * CH + k * 16
            p = base + off + iota
            q = (p.astype(jnp.float32) * inv_l).astype(jnp.int32)
            q = jnp.where((q + 1) * L <= p, q + 1, q)
            q = jnp.where(q * L > p, q - 1, q)
            l = p - q * L
            seq = plsc.load_gather(seq_v, [q])
            valid = l < seq
            v01 = jnp.where(valid, 1, 0)
            csum = plsc.cumsum(v01)
            cntv = jnp.sum(v01)
            tokvec = tok_in[pl.ds(off, 16)]
            vidx = nv + csum - 1
            plsc.store_scatter(vtok, [vidx], tokvec, mask=valid)
            plsc.store_scatter(vdst, [vidx >> 7, vidx & (CH - 1)], p,
                               mask=valid)
            zidx = ni + (iota + 1 - csum) - 1
            plsc.store_scatter(zdst, [zidx >> 7, zidx & (CH - 1)], p,
                               mask=jnp.logical_not(valid))
            nv = nv + cntv
            ni = ni + (16 - cntv)
        return nv, ni

    nv, ni = lax.fori_loop(0, NCH, compute_chunk, (jnp.int32(0), jnp.int32(0)))

    iv = jnp.full((16,), 1, jnp.int32) * jnp.maximum(nv - 1, 0)
    tok_last = plsc.load_gather(vtok, [iv])
    dst_last = plsc.load_gather(vdst, [iv >> 7, iv & (CH - 1)])
    iz = jnp.full((16,), 1, jnp.int32) * jnp.maximum(ni - 1, 0)
    z_last = plsc.load_gather(zdst, [iz >> 7, iz & (CH - 1)])
    nv_al = nv & ~15
    ni_al = ni & ~15
    for k in range(9):
        pv = nv_al + k * 16 + iota
        mv = pv >= nv
        plsc.store_scatter(vtok, [pv], tok_last, mask=mv)
        plsc.store_scatter(vdst, [pv >> 7, pv & (CH - 1)], dst_last, mask=mv)
        pz = ni_al + k * 16 + iota
        plsc.store_scatter(zdst, [pz >> 7, pz & (CH - 1)], z_last,
                           mask=pz >= ni)

    nchv = (nv + CH - 1) >> 7
    nchz = (ni + CH - 1) >> 7

    def fire_zero(z, _):
        pltpu.async_copy(zero_v, out_hbm.at[zdst.at[z]], sem_z)
        return 0
    lax.fori_loop(0, nchz, fire_zero, 0)

    def fire_gather(g, slot):
        pltpu.async_copy(
            table_hbm.at[vtok.at[pl.ds(g * CH, CH)]], rows_v.at[slot], sem_g)

    @pl.when(nchv > 0)
    def _prime():
        fire_gather(0, 0)

    def move_chunk(g, _):
        slot = g & 1
        pltpu.make_async_copy(
            table_hbm.at[vtok.at[pl.ds(g * CH, CH)]], rows_v.at[slot],
            sem_g).wait()

        @pl.when(g + 1 < nchv)
        def _next():
            fire_gather(g + 1, (g + 1) & 1)

        pltpu.async_copy(rows_v.at[slot], out_hbm.at[vdst.at[g]],
                         sem_s).wait()
        return 0

    lax.fori_loop(0, nchv, move_chunk, 0)

    def drain_zero(z, _):
        pltpu.make_async_copy(zero_v, out_hbm.at[zdst.at[0]], sem_z).wait()
        return 0
    lax.fori_loop(0, nchz, drain_zero, 0)


def kernel(token_ids, seq_lens, glove_table):
    tok_flat = token_ids.reshape(B * L).astype(jnp.int32)
    seq_lens = seq_lens.astype(jnp.int32)
    zeros_src = jnp.zeros((CH, D), jnp.float32)

    mesh = plsc.VectorSubcoreMesh(core_axis_name="c", subcore_axis_name="s")
    out = pl.kernel(
        _body,
        out_type=jax.ShapeDtypeStruct((B * L, D), jnp.float32),
        mesh=mesh,
        compiler_params=pltpu.CompilerParams(use_tc_tiling_on_sc=False,
                                             needs_layout_passes=False),
        scratch_types=[
            pltpu.VMEM((PPW,), jnp.int32),
            pltpu.VMEM((B,), jnp.int32),
            pltpu.VMEM((PPW + 144,), jnp.int32),
            pltpu.VMEM((NCH + 2, CH), jnp.int32),
            pltpu.VMEM((NCH + 2, CH), jnp.int32),
            pltpu.VMEM((2, CH, D), jnp.float32),
            pltpu.VMEM((CH, D), jnp.float32),
            pltpu.SemaphoreType.DMA,
            pltpu.SemaphoreType.DMA,
            pltpu.SemaphoreType.DMA,
        ],
    )(tok_flat, seq_lens, glove_table, zeros_src)
    return out.reshape(B, L, D)
